# baseline (speedup 1.0000x reference)
import functools

import jax
import jax.numpy as jnp
from jax import lax
from jax.experimental import pallas as pl
from jax.experimental.pallas import tpu as pltpu
from jax.experimental.pallas import tpu_sc as plsc

_B, _L, _D = 64, 2048, 300
_DP = 304
_N = _B * _L
_NC, _NS = 2, 16
_NW = _NC * _NS
_TPW = _N // _NW
_C = 32
_NCHUNK = _TPW // _C
_NBLK = _N // _C
_LANES = 16
_NGRP = 19

_mesh = plsc.VectorSubcoreMesh(core_axis_name="c", subcore_axis_name="s")


@functools.partial(
    pl.kernel,
    out_type=jax.ShapeDtypeStruct((_N, _D), jnp.float32),
    mesh=_mesh,
    compiler_params=pltpu.CompilerParams(use_tc_tiling_on_sc=False),
    scratch_types=[
        pltpu.VMEM((3, 5, _C), jnp.int32),
        pltpu.VMEM((3, _C, _D), jnp.float32),
        pltpu.VMEM((2, _C, _DP), jnp.float32),
        pltpu.VMEM((2, _C, _DP), jnp.float32),
        pltpu.VMEM((2, _C, _DP), jnp.float32),
        pltpu.VMEM((2, _C, _DP), jnp.float32),
        pltpu.VMEM((2, _C, _DP), jnp.float32),
        pltpu.SemaphoreType.DMA,
        pltpu.SemaphoreType.DMA,
        pltpu.SemaphoreType.DMA,
        pltpu.SemaphoreType.DMA,
        pltpu.SemaphoreType.DMA,
        pltpu.SemaphoreType.DMA,
        pltpu.SemaphoreType.DMA,
        pltpu.SemaphoreType.DMA,
        pltpu.SemaphoreType.DMA,
        pltpu.SemaphoreType.DMA,
        pltpu.SemaphoreType.DMA,
    ],
)
def _embed_sum(tok_hbm, idx_hbm,
               pos_hbm, seg_hbm, col_hbm, row_hbm, rank_hbm, out_hbm,
               idxb, tbuf, g_p, g_s, g_c, g_r, g_k,
               isem0, isem1, isem2, tsem0, tsem1, tsem2,
               gsem0, gsem1, ssem0, ssem1, ssem2):
  wid = lax.axis_index("s") * _NC + lax.axis_index("c")
  base0 = wid * _TPW
  blk0 = wid * _NCHUNK
  isems = [isem0, isem1, isem2]
  tsems = [tsem0, tsem1, tsem2]
  gsems = [gsem0, gsem1]
  ssems = [ssem0, ssem1, ssem2]
  gbufs = [g_p, g_s, g_c, g_r, g_k]
  tabs = [pos_hbm, seg_hbm, col_hbm, row_hbm, rank_hbm]

  def fire_loads(c, s3):
    pltpu.async_copy(idx_hbm.at[blk0 + c], idxb.at[s3], isems[s3])
    pltpu.async_copy(tok_hbm.at[pl.ds(base0 + c * _C, _C)], tbuf.at[s3],
                     tsems[s3])

  def wait_loads(s3):
    pltpu.make_async_copy(idx_hbm.at[blk0], idxb.at[s3], isems[s3]).wait()

  def fire_gathers(s3, g2):
    for t in range(5):
      pltpu.async_copy(tabs[t].at[idxb.at[s3, t]], gbufs[t].at[g2], gsems[g2])

  def wait_gathers_tok(s3, g2):
    pltpu.make_async_copy(tok_hbm.at[pl.ds(base0, _C)], tbuf.at[s3],
                          tsems[s3]).wait()
    for t in range(5):
      pltpu.make_async_copy(tabs[t].at[idxb.at[s3, t]], gbufs[t].at[g2],
                            gsems[g2]).wait()

  def compute(s3, g2):
    def row_body(i, carry2):
      vals = []
      for j in range(_NGRP):
        d = _D - _LANES if j == _NGRP - 1 else _LANES * j
        v = tbuf[s3, i, pl.ds(d, _LANES)]
        v = v + g_p[g2, i, pl.ds(d, _LANES)]
        v = v + g_s[g2, i, pl.ds(d, _LANES)]
        v = v + g_c[g2, i, pl.ds(d, _LANES)]
        v = v + g_r[g2, i, pl.ds(d, _LANES)]
        v = v + g_k[g2, i, pl.ds(d, _LANES)]
        vals.append(v)
      for j in range(_NGRP):
        d = _D - _LANES if j == _NGRP - 1 else _LANES * j
        tbuf[s3, i, pl.ds(d, _LANES)] = vals[j]
      return carry2

    lax.fori_loop(0, _C, row_body, 0, unroll=False)

  def fire_store(c, s3):
    pltpu.async_copy(tbuf.at[s3], out_hbm.at[pl.ds(base0 + c * _C, _C)],
                     ssems[s3])

  def wait_store(s3):
    pltpu.make_async_copy(tbuf.at[s3], out_hbm.at[pl.ds(base0, _C)],
                          ssems[s3]).wait()

  fire_loads(0, 0)
  fire_loads(1, 1)
  wait_loads(0)
  fire_gathers(0, 0)

  fire_loads(2, 2)
  wait_loads(1)
  fire_gathers(1, 1)
  wait_gathers_tok(0, 0)
  compute(0, 0)
  fire_store(0, 0)

  def steady6(m, carry):
    c0 = 1 + m * 6
    for k in range(6):
      ck = c0 + k
      s3 = (1 + k) % 3
      s3n = (2 + k) % 3
      s3nn = (3 + k) % 3
      g2 = (1 + k) % 2
      g2n = (2 + k) % 2
      wait_store(s3nn)
      fire_loads(ck + 2, s3nn)
      wait_loads(s3n)
      fire_gathers(s3n, g2n)
      wait_gathers_tok(s3, g2)
      compute(s3, g2)
      fire_store(ck, s3)
    return carry

  lax.fori_loop(0, 20, steady6, 0, unroll=False)

  for ck in range(121, _NCHUNK):
    s3 = ck % 3
    s3n = (ck + 1) % 3
    s3nn = (ck + 2) % 3
    g2 = ck % 2
    g2n = (ck + 1) % 2
    if ck + 2 < _NCHUNK:
      wait_store(s3nn)
      fire_loads(ck + 2, s3nn)
    if ck + 1 < _NCHUNK:
      wait_loads(s3n)
      fire_gathers(s3n, g2n)
    wait_gathers_tok(s3, g2)
    compute(s3, g2)
    fire_store(ck, s3)

  wait_store((_NCHUNK - 3) % 3)
  wait_store((_NCHUNK - 2) % 3)
  wait_store((_NCHUNK - 1) % 3)


def _pad(t):
  return jnp.pad(t, ((0, 0), (0, _DP - _D)))


def kernel(token_vecs, pos_idx, seg_idx, col_idx, row_idx, rank_idx,
           pos, seg_id, col_id, row_id, rank_id):
  idx = jnp.stack([pos_idx.reshape(_N), seg_idx.reshape(_N),
                   col_idx.reshape(_N), row_idx.reshape(_N),
                   rank_idx.reshape(_N)], axis=0)
  idx_blocks = idx.reshape(5, _NBLK, _C).transpose(1, 0, 2)
  out = _embed_sum(
      token_vecs.reshape(_N, _D), idx_blocks,
      _pad(pos), _pad(seg_id), _pad(col_id), _pad(row_id), _pad(rank_id))
  return out.reshape(_B, _L, _D)


# speedup vs baseline: 1.0110x; 1.0110x over previous
import functools

import jax
import jax.numpy as jnp
from jax import lax
from jax.experimental import pallas as pl
from jax.experimental.pallas import tpu as pltpu
from jax.experimental.pallas import tpu_sc as plsc

_B, _L, _D = 64, 2048, 300
_DP = 304
_N = _B * _L
_NC, _NS = 2, 16
_NW = _NC * _NS
_TPW = _N // _NW
_C = 64
_NCHUNK = _TPW // _C
_NBLK = _N // _C
_LANES = 16
_NGRP = 19

_mesh = plsc.VectorSubcoreMesh(core_axis_name="c", subcore_axis_name="s")


@functools.partial(
    pl.kernel,
    out_type=jax.ShapeDtypeStruct((_N, _D), jnp.float32),
    mesh=_mesh,
    compiler_params=pltpu.CompilerParams(use_tc_tiling_on_sc=False),
    scratch_types=[
        pltpu.VMEM((4, 5, _C), jnp.int32),
        pltpu.VMEM((3, _C, _DP), jnp.float32),
        pltpu.VMEM((3, _C, _D), jnp.float32),
        pltpu.SemaphoreType.DMA,
        pltpu.SemaphoreType.DMA,
        pltpu.SemaphoreType.DMA,
        pltpu.SemaphoreType.DMA,
        pltpu.SemaphoreType.DMA,
        pltpu.SemaphoreType.DMA,
        pltpu.SemaphoreType.DMA,
        pltpu.SemaphoreType.DMA,
        pltpu.SemaphoreType.DMA,
        pltpu.SemaphoreType.DMA,
        pltpu.SemaphoreType.DMA,
        pltpu.SemaphoreType.DMA,
        pltpu.SemaphoreType.DMA,
        pltpu.SemaphoreType.DMA,
        pltpu.SemaphoreType.DMA,
        pltpu.SemaphoreType.DMA,
    ],
)
def _embed_sum(tok_hbm, idx_hbm,
               pos_hbm, seg_hbm, col_hbm, row_hbm, rank_hbm, out_hbm,
               idxb, acc, tbuf,
               isem0, isem1, isem2, isem3,
               psem0, psem1, psem2,
               asem0, asem1, asem2,
               tsem0, tsem1, tsem2,
               ssem0, ssem1, ssem2):
  wid = lax.axis_index("s") * _NC + lax.axis_index("c")
  base0 = wid * _TPW
  blk0 = wid * _NCHUNK
  isems = [isem0, isem1, isem2, isem3]
  psems = [psem0, psem1, psem2]
  asems = [asem0, asem1, asem2]
  tsems = [tsem0, tsem1, tsem2]
  ssems = [ssem0, ssem1, ssem2]
  addtabs = [seg_hbm, col_hbm, row_hbm, rank_hbm]

  def fire_idx(c, s4):
    pltpu.async_copy(idx_hbm.at[blk0 + c], idxb.at[s4], isems[s4])

  def wait_idx(s4):
    pltpu.make_async_copy(idx_hbm.at[blk0], idxb.at[s4], isems[s4]).wait()

  def fire_pos_tok(c, s4, s3):
    pltpu.async_copy(pos_hbm.at[idxb.at[s4, 0]], acc.at[s3], psems[s3])
    pltpu.async_copy(tok_hbm.at[pl.ds(base0 + c * _C, _C)], tbuf.at[s3],
                     tsems[s3])

  def wait_pos(s4, s3):
    pltpu.make_async_copy(pos_hbm.at[idxb.at[s4, 0]], acc.at[s3],
                          psems[s3]).wait()

  def fire_adds(s4, s3):
    for t in range(4):
      pltpu.async_copy(addtabs[t].at[idxb.at[s4, t + 1]], acc.at[s3],
                       asems[s3], add=True)

  def wait_adds_tok(s4, s3):
    pltpu.make_async_copy(tok_hbm.at[pl.ds(base0, _C)], tbuf.at[s3],
                          tsems[s3]).wait()
    for t in range(4):
      pltpu.make_async_copy(addtabs[t].at[idxb.at[s4, t + 1]], acc.at[s3],
                            asems[s3]).wait()

  def compute(s3):
    def row_body(i, carry2):
      vals = []
      for j in range(_NGRP):
        d = _D - _LANES if j == _NGRP - 1 else _LANES * j
        v = tbuf[s3, i, pl.ds(d, _LANES)] + acc[s3, i, pl.ds(d, _LANES)]
        vals.append(v)
      for j in range(_NGRP):
        d = _D - _LANES if j == _NGRP - 1 else _LANES * j
        tbuf[s3, i, pl.ds(d, _LANES)] = vals[j]
      return carry2

    lax.fori_loop(0, _C, row_body, 0, unroll=False)

  def fire_store(c, s3):
    pltpu.async_copy(tbuf.at[s3], out_hbm.at[pl.ds(base0 + c * _C, _C)],
                     ssems[s3])

  def wait_store(s3):
    pltpu.make_async_copy(tbuf.at[s3], out_hbm.at[pl.ds(base0, _C)],
                          ssems[s3]).wait()

  fire_idx(0, 0)
  fire_idx(1, 1)
  fire_idx(2, 2)
  wait_idx(0)
  fire_pos_tok(0, 0, 0)
  wait_idx(1)
  fire_pos_tok(1, 1, 1)
  wait_pos(0, 0)
  fire_adds(0, 0)

  def chunk(c, s4, s3, s4n, s3n, s4nn, s3nn,
            do_idx, do_adds_next, do_loads_next2, do_store_wait):
    if do_idx:
      fire_idx(c + 3, (s4 + 3) % 4)
    if do_adds_next:
      wait_pos(s4n, s3n)
      fire_adds(s4n, s3n)
    wait_adds_tok(s4, s3)
    compute(s3)
    fire_store(c, s3)
    if do_loads_next2:
      if do_store_wait:
        wait_store(s3nn)
      wait_idx(s4nn)
      fire_pos_tok(c + 2, s4nn, s3nn)

  chunk(0, 0, 0, 1, 1, 2, 2, True, True, True, False)

  def steady12(m, carry):
    c0 = 1 + m * 12
    for k in range(12):
      ck = c0 + k
      s4 = (1 + k) % 4
      s3 = (1 + k) % 3
      chunk(ck, s4, s3, (s4 + 1) % 4, (s3 + 1) % 3, (s4 + 2) % 4,
            (s3 + 2) % 3, True, True, True, True)
    return carry

  lax.fori_loop(0, 5, steady12, 0, unroll=False)

  chunk(61, 61 % 4, 61 % 3, 62 % 4, 62 % 3, 63 % 4, 63 % 3,
        False, True, True, True)
  chunk(62, 62 % 4, 62 % 3, 63 % 4, 63 % 3, 0, 0, False, True, False, False)
  chunk(63, 63 % 4, 63 % 3, 0, 0, 0, 0, False, False, False, False)

  wait_store(61 % 3)
  wait_store(62 % 3)
  wait_store(63 % 3)


def _pad(t):
  return jnp.pad(t, ((0, 0), (0, _DP - _D)))


def kernel(token_vecs, pos_idx, seg_idx, col_idx, row_idx, rank_idx,
           pos, seg_id, col_id, row_id, rank_id):
  idx = jnp.stack([pos_idx.reshape(_N), seg_idx.reshape(_N),
                   col_idx.reshape(_N), row_idx.reshape(_N),
                   rank_idx.reshape(_N)], axis=0)
  idx_blocks = idx.reshape(5, _NBLK, _C).transpose(1, 0, 2)
  out = _embed_sum(
      token_vecs.reshape(_N, _D), idx_blocks,
      _pad(pos), _pad(seg_id), _pad(col_id), _pad(row_id), _pad(rank_id))
  return out.reshape(_B, _L, _D)


# speedup vs baseline: 3.2944x; 3.2586x over previous
import functools

import jax
import jax.numpy as jnp
from jax import lax
from jax.experimental import pallas as pl
from jax.experimental.pallas import tpu as pltpu
from jax.experimental.pallas import tpu_sc as plsc

_B, _L, _D = 64, 2048, 300
_DP = 304
_N = _B * _L
_NC, _NS = 2, 16
_NW = _NC * _NS
_TPW = _N // _NW
_C = 64
_NCHUNK = _TPW // _C
_NBLK = _N // _C
_LANES = 16
_NGRP = 19

_mesh = plsc.VectorSubcoreMesh(core_axis_name="c", subcore_axis_name="s")


@functools.partial(
    pl.kernel,
    out_type=jax.ShapeDtypeStruct((_N, _D), jnp.float32),
    mesh=_mesh,
    compiler_params=pltpu.CompilerParams(use_tc_tiling_on_sc=False, needs_layout_passes=False),
    scratch_types=[
        pltpu.VMEM((4, 5, _C), jnp.int32),
        pltpu.VMEM((3, _C, _DP), jnp.float32),
        pltpu.VMEM((3, _C, _D), jnp.float32),
        pltpu.VMEM((1, _DP), jnp.float32),
        pltpu.SemaphoreType.DMA,
        pltpu.SemaphoreType.DMA,
        pltpu.SemaphoreType.DMA,
        pltpu.SemaphoreType.DMA,
        pltpu.SemaphoreType.DMA,
        pltpu.SemaphoreType.DMA,
        pltpu.SemaphoreType.DMA,
        pltpu.SemaphoreType.DMA,
        pltpu.SemaphoreType.DMA,
        pltpu.SemaphoreType.DMA,
        pltpu.SemaphoreType.DMA,
        pltpu.SemaphoreType.DMA,
        pltpu.SemaphoreType.DMA,
        pltpu.SemaphoreType.DMA,
        pltpu.SemaphoreType.DMA,
        pltpu.SemaphoreType.DMA,
    ],
)
def _embed_sum(tok_hbm, idx_hbm,
               pos_hbm, diff_hbm, col_hbm, row_hbm, rank_hbm, out_hbm,
               idxb, acc, tbuf, dbuf,
               isem0, isem1, isem2, isem3,
               psem0, psem1, psem2,
               asem0, asem1, asem2,
               tsem0, tsem1, tsem2,
               ssem0, ssem1, ssem2):
  wid = lax.axis_index("s") * _NC + lax.axis_index("c")
  base0 = wid * _TPW
  blk0 = wid * _NCHUNK
  isems = [isem0, isem1, isem2, isem3]
  psems = [psem0, psem1, psem2]
  asems = [asem0, asem1, asem2]
  tsems = [tsem0, tsem1, tsem2]
  ssems = [ssem0, ssem1, ssem2]
  addtabs = [col_hbm, row_hbm, rank_hbm]

  def fire_idx(c, s4):
    pltpu.async_copy(idx_hbm.at[blk0 + c], idxb.at[s4], isems[s4])

  def wait_idx(s4):
    pltpu.make_async_copy(idx_hbm.at[blk0], idxb.at[s4], isems[s4]).wait()

  def fire_pos_tok(c, s4, s3):
    pltpu.async_copy(pos_hbm.at[idxb.at[s4, 0]], acc.at[s3], psems[s3])
    pltpu.async_copy(tok_hbm.at[pl.ds(base0 + c * _C, _C)], tbuf.at[s3],
                     tsems[s3])

  def wait_pos(s4, s3):
    pltpu.make_async_copy(pos_hbm.at[idxb.at[s4, 0]], acc.at[s3],
                          psems[s3]).wait()

  def fire_adds(s4, s3):
    for t in range(3):
      pltpu.async_copy(addtabs[t].at[idxb.at[s4, t + 2]], acc.at[s3],
                       asems[s3], add=True)

  def wait_adds_tok(s4, s3):
    pltpu.make_async_copy(tok_hbm.at[pl.ds(base0, _C)], tbuf.at[s3],
                          tsems[s3]).wait()
    for t in range(3):
      pltpu.make_async_copy(addtabs[t].at[idxb.at[s4, t + 2]], acc.at[s3],
                            asems[s3]).wait()

  def compute(s4, s3):
    @plsc.parallel_loop(0, _C, step=1, unroll=2)
    def row_body(i):
      s_vec = plsc.load_gather(
          idxb, [jnp.full((_LANES,), s4, jnp.int32),
                 jnp.full((_LANES,), 1, jnp.int32),
                 jnp.full((_LANES,), i, jnp.int32)])
      s_f = s_vec.astype(jnp.float32)
      vals = []
      for j in range(_NGRP):
        d = _D - _LANES if j == _NGRP - 1 else _LANES * j
        v = tbuf[s3, i, pl.ds(d, _LANES)] + acc[s3, i, pl.ds(d, _LANES)]
        v = v + s_f * dbuf[0, pl.ds(d, _LANES)]
        vals.append(v)
      for j in range(_NGRP):
        d = _D - _LANES if j == _NGRP - 1 else _LANES * j
        tbuf[s3, i, pl.ds(d, _LANES)] = vals[j]

  def fire_store(c, s3):
    pltpu.async_copy(tbuf.at[s3], out_hbm.at[pl.ds(base0 + c * _C, _C)],
                     ssems[s3])

  def wait_store(s3):
    pltpu.make_async_copy(tbuf.at[s3], out_hbm.at[pl.ds(base0, _C)],
                          ssems[s3]).wait()

  pltpu.sync_copy(diff_hbm, dbuf)
  fire_idx(0, 0)
  fire_idx(1, 1)
  fire_idx(2, 2)
  wait_idx(0)
  fire_pos_tok(0, 0, 0)
  wait_idx(1)
  fire_pos_tok(1, 1, 1)
  wait_pos(0, 0)
  fire_adds(0, 0)

  def chunk(c, s4, s3, s4n, s3n, s4nn, s3nn,
            do_idx, do_adds_next, do_loads_next2, do_store_wait):
    if do_idx:
      fire_idx(c + 3, (s4 + 3) % 4)
    if do_adds_next:
      wait_pos(s4n, s3n)
      fire_adds(s4n, s3n)
    wait_adds_tok(s4, s3)
    compute(s4, s3)
    fire_store(c, s3)
    if do_loads_next2:
      if do_store_wait:
        wait_store(s3nn)
      wait_idx(s4nn)
      fire_pos_tok(c + 2, s4nn, s3nn)

  chunk(0, 0, 0, 1, 1, 2, 2, True, True, True, False)

  def steady12(m, carry):
    c0 = 1 + m * 12
    for k in range(12):
      ck = c0 + k
      s4 = (1 + k) % 4
      s3 = (1 + k) % 3
      chunk(ck, s4, s3, (s4 + 1) % 4, (s3 + 1) % 3, (s4 + 2) % 4,
            (s3 + 2) % 3, True, True, True, True)
    return carry

  lax.fori_loop(0, 5, steady12, 0, unroll=False)

  chunk(61, 61 % 4, 61 % 3, 62 % 4, 62 % 3, 63 % 4, 63 % 3,
        False, True, True, True)
  chunk(62, 62 % 4, 62 % 3, 63 % 4, 63 % 3, 0, 0, False, True, False, False)
  chunk(63, 63 % 4, 63 % 3, 0, 0, 0, 0, False, False, False, False)

  wait_store(61 % 3)
  wait_store(62 % 3)
  wait_store(63 % 3)


def _pad(t):
  return jnp.pad(t, ((0, 0), (0, _DP - _D)))


def kernel(token_vecs, pos_idx, seg_idx, col_idx, row_idx, rank_idx,
           pos, seg_id, col_id, row_id, rank_id):
  idx = jnp.stack([pos_idx.reshape(_N), seg_idx.reshape(_N),
                   col_idx.reshape(_N), row_idx.reshape(_N),
                   rank_idx.reshape(_N)], axis=0)
  idx_blocks = idx.reshape(5, _NBLK, _C).transpose(1, 0, 2)
  pos_eff = pos + seg_id[0][None, :]
  diff = (seg_id[1] - seg_id[0])[None, :]
  out = _embed_sum(
      token_vecs.reshape(_N, _D), idx_blocks,
      _pad(pos_eff), _pad(diff), _pad(col_id), _pad(row_id), _pad(rank_id))
  return out.reshape(_B, _L, _D)


# speedup vs baseline: 3.9207x; 1.1901x over previous
import functools

import jax
import jax.numpy as jnp
from jax import lax
from jax.experimental import pallas as pl
from jax.experimental.pallas import tpu as pltpu
from jax.experimental.pallas import tpu_sc as plsc

_B, _L, _D = 64, 2048, 300
_DP = 304
_N = _B * _L
_NC, _NS = 2, 16
_NW = _NC * _NS
_TPW = _N // _NW
_C = 64
_NCHUNK = _TPW // _C
_NBLK = _N // _C
_LANES = 16
_NGRP = 19

_mesh = plsc.VectorSubcoreMesh(core_axis_name="c", subcore_axis_name="s")


@functools.partial(
    pl.kernel,
    out_type=jax.ShapeDtypeStruct((_N, _DP), jnp.float32),
    mesh=_mesh,
    compiler_params=pltpu.CompilerParams(use_tc_tiling_on_sc=False, needs_layout_passes=False),
    scratch_types=[
        pltpu.VMEM((4, 5, _C), jnp.int32),
        pltpu.VMEM((3, _C, _DP), jnp.float32),
        pltpu.SemaphoreType.DMA,
        pltpu.SemaphoreType.DMA,
        pltpu.SemaphoreType.DMA,
        pltpu.SemaphoreType.DMA,
        pltpu.SemaphoreType.DMA,
        pltpu.SemaphoreType.DMA,
        pltpu.SemaphoreType.DMA,
        pltpu.SemaphoreType.DMA,
        pltpu.SemaphoreType.DMA,
        pltpu.SemaphoreType.DMA,
        pltpu.SemaphoreType.DMA,
        pltpu.SemaphoreType.DMA,
        pltpu.SemaphoreType.DMA,
    ],
)
def _embed_sum(idx_hbm,
               pos_hbm, col_hbm, row_hbm, rank_hbm, out_hbm,
               idxb, acc,
               isem0, isem1, isem2, isem3,
               psem0, psem1, psem2,
               asem0, asem1, asem2,
               ssem0, ssem1, ssem2):
  wid = lax.axis_index("s") * _NC + lax.axis_index("c")
  base0 = wid * _TPW
  blk0 = wid * _NCHUNK
  isems = [isem0, isem1, isem2, isem3]
  psems = [psem0, psem1, psem2]
  asems = [asem0, asem1, asem2]
  ssems = [ssem0, ssem1, ssem2]
  addtabs = [col_hbm, row_hbm, rank_hbm]

  def fire_idx(c, s4):
    pltpu.async_copy(idx_hbm.at[blk0 + c], idxb.at[s4], isems[s4])

  def wait_idx(s4):
    pltpu.make_async_copy(idx_hbm.at[blk0], idxb.at[s4], isems[s4]).wait()

  def fire_pos_tok(c, s4, s3):
    pltpu.async_copy(pos_hbm.at[idxb.at[s4, 0]], acc.at[s3], psems[s3])

  def wait_pos(s4, s3):
    pltpu.make_async_copy(pos_hbm.at[idxb.at[s4, 0]], acc.at[s3],
                          psems[s3]).wait()

  def fire_adds(s4, s3):
    for t in range(3):
      pltpu.async_copy(addtabs[t].at[idxb.at[s4, t + 2]], acc.at[s3],
                       asems[s3], add=True)

  def wait_adds_tok(s4, s3):
    for t in range(3):
      pltpu.make_async_copy(addtabs[t].at[idxb.at[s4, t + 2]], acc.at[s3],
                            asems[s3]).wait()

  def fire_store(c, s3):
    pltpu.async_copy(acc.at[s3], out_hbm.at[pl.ds(base0 + c * _C, _C)],
                     ssems[s3])

  def wait_store(s3):
    pltpu.make_async_copy(acc.at[s3], out_hbm.at[pl.ds(base0, _C)],
                          ssems[s3]).wait()

  fire_idx(0, 0)
  fire_idx(1, 1)
  fire_idx(2, 2)
  wait_idx(0)
  fire_pos_tok(0, 0, 0)
  wait_idx(1)
  fire_pos_tok(1, 1, 1)
  wait_pos(0, 0)
  fire_adds(0, 0)

  def chunk(c, s4, s3, s4n, s3n, s4nn, s3nn,
            do_idx, do_adds_next, do_loads_next2, do_store_wait):
    if do_idx:
      fire_idx(c + 3, (s4 + 3) % 4)
    if do_adds_next:
      wait_pos(s4n, s3n)
      fire_adds(s4n, s3n)
    wait_adds_tok(s4, s3)
    fire_store(c, s3)
    if do_loads_next2:
      if do_store_wait:
        wait_store(s3nn)
      wait_idx(s4nn)
      fire_pos_tok(c + 2, s4nn, s3nn)

  chunk(0, 0, 0, 1, 1, 2, 2, True, True, True, False)

  def steady12(m, carry):
    c0 = 1 + m * 12
    for k in range(12):
      ck = c0 + k
      s4 = (1 + k) % 4
      s3 = (1 + k) % 3
      chunk(ck, s4, s3, (s4 + 1) % 4, (s3 + 1) % 3, (s4 + 2) % 4,
            (s3 + 2) % 3, True, True, True, True)
    return carry

  lax.fori_loop(0, 5, steady12, 0, unroll=False)

  chunk(61, 61 % 4, 61 % 3, 62 % 4, 62 % 3, 63 % 4, 63 % 3,
        False, True, True, True)
  chunk(62, 62 % 4, 62 % 3, 63 % 4, 63 % 3, 0, 0, False, True, False, False)
  chunk(63, 63 % 4, 63 % 3, 0, 0, 0, 0, False, False, False, False)

  wait_store(61 % 3)
  wait_store(62 % 3)
  wait_store(63 % 3)


def _pad(t):
  return jnp.pad(t, ((0, 0), (0, _DP - _D)))


_RN = 1024


def _combine_body(tok_ref, sums_ref, segf_ref, diff_ref, o_ref):
  o_ref[...] = (tok_ref[...] + sums_ref[:, :_D]
                + segf_ref[...] * diff_ref[...])


_combine = pl.pallas_call(
    _combine_body,
    out_shape=jax.ShapeDtypeStruct((_N, _D), jnp.float32),
    grid=(_N // _RN,),
    in_specs=[
        pl.BlockSpec((_RN, _D), lambda i: (i, 0)),
        pl.BlockSpec((_RN, _DP), lambda i: (i, 0)),
        pl.BlockSpec((_RN, 1), lambda i: (i, 0)),
        pl.BlockSpec((1, _D), lambda i: (0, 0)),
    ],
    out_specs=pl.BlockSpec((_RN, _D), lambda i: (i, 0)),
)


def kernel(token_vecs, pos_idx, seg_idx, col_idx, row_idx, rank_idx,
           pos, seg_id, col_id, row_id, rank_id):
  idx = jnp.stack([pos_idx.reshape(_N), seg_idx.reshape(_N),
                   col_idx.reshape(_N), row_idx.reshape(_N),
                   rank_idx.reshape(_N)], axis=0)
  idx_blocks = idx.reshape(5, _NBLK, _C).transpose(1, 0, 2)
  pos_eff = pos + seg_id[0][None, :]
  diff = (seg_id[1] - seg_id[0])[None, :]
  sums = _embed_sum(
      idx_blocks,
      _pad(pos_eff), _pad(col_id), _pad(row_id), _pad(rank_id))
  out = _combine(token_vecs.reshape(_N, _D), sums,
                 seg_idx.reshape(_N, 1).astype(jnp.float32), diff)
  return out.reshape(_B, _L, _D)
